# baseline (speedup 1.0000x reference)
import functools

import jax
import jax.numpy as jnp
from jax import lax
from jax.experimental import pallas as pl
from jax.experimental.pallas import tpu as pltpu
from jax.experimental.pallas import tpu_sc as plsc


def _mixture_select(probs, p, params_flat, K, B, D, NC):
    mesh = plsc.ScalarSubcoreMesh(axis_name="c", num_cores=NC)
    rows_per = B // NC

    @functools.partial(
        pl.kernel,
        out_type=jax.ShapeDtypeStruct((B, D), jnp.float32),
        mesh=mesh,
        compiler_params=pltpu.CompilerParams(skip_device_barrier=True),
        scratch_types=[
            pltpu.SMEM((K,), jnp.float32),
            pltpu.SMEM((1,), jnp.float32),
            pltpu.VMEM_SHARED((rows_per, D), jnp.float32),
            pltpu.SemaphoreType.DMA,
            pltpu.SemaphoreType.DMA,
            pltpu.SemaphoreType.DMA,
        ],
    )
    def run(probs_hbm, p_hbm, params_hbm, out_hbm,
            probs_s, p_s, stage, selsem, gsem, ssem):
        cid = lax.axis_index("c")
        c1 = pltpu.async_copy(probs_hbm, probs_s, selsem)
        c2 = pltpu.async_copy(p_hbm, p_s, selsem)
        c1.wait()
        c2.wait()
        pv = p_s[0]
        acc = jnp.float32(0.0)
        idx = jnp.int32(0)
        for k in range(K):
            acc = acc + probs_s[k]
            idx = idx + jnp.where(acc < pv, jnp.int32(1), jnp.int32(0))
        idx = jnp.minimum(idx, jnp.int32(K - 1))
        src_row = idx * B + cid * rows_per
        dst_row = cid * rows_per
        pltpu.async_copy(
            params_hbm.at[pl.ds(src_row, rows_per)],
            out_hbm.at[pl.ds(dst_row, rows_per)],
            gsem,
        ).wait()

    return run(probs, p, params_flat)


def kernel(probs, p, params, batch_size):
    K, B, D = params.shape
    info = plsc.get_sparse_core_info()
    NC = info.num_cores
    params_flat = params.reshape(K * B, D)
    return _mixture_select(probs, p, params_flat, K, B, D, NC)


# speedup vs baseline: 3.8917x; 3.8917x over previous
import functools

import jax
import jax.numpy as jnp
from jax import lax
from jax.experimental import pallas as pl
from jax.experimental.pallas import tpu as pltpu
from jax.experimental.pallas import tpu_sc as plsc


def _mixture_select(probs, p, params_flat, K, B, D, NC):
    mesh = plsc.ScalarSubcoreMesh(axis_name="c", num_cores=NC)
    rows_per = B // NC

    @functools.partial(
        pl.kernel,
        out_type=jax.ShapeDtypeStruct((B, D), jnp.float32),
        mesh=mesh,
        compiler_params=pltpu.CompilerParams(skip_device_barrier=True),
        scratch_types=[
            pltpu.SMEM((K,), jnp.float32),
            pltpu.SMEM((1,), jnp.float32),
            pltpu.VMEM_SHARED((rows_per, D), jnp.float32),
            pltpu.SemaphoreType.DMA,
            pltpu.SemaphoreType.DMA,
            pltpu.SemaphoreType.DMA,
        ],
    )
    def run(probs_hbm, p_hbm, params_hbm, out_hbm,
            probs_s, p_s, stage, selsem, gsem, ssem):
        cid = lax.axis_index("c")
        c1 = pltpu.async_copy(probs_hbm, probs_s, selsem)
        c2 = pltpu.async_copy(p_hbm, p_s, selsem)
        c1.wait()
        c2.wait()
        pv = p_s[0]
        acc = jnp.float32(0.0)
        idx = jnp.int32(0)
        for k in range(K):
            acc = acc + probs_s[k]
            idx = idx + jnp.where(acc < pv, jnp.int32(1), jnp.int32(0))
        idx = jnp.minimum(idx, jnp.int32(K - 1))
        src_row = idx * B + cid * rows_per
        dst_row = cid * rows_per
        nchunk = 8
        rc = rows_per // nchunk
        gathers = []
        for i in range(nchunk):
            gathers.append(
                pltpu.async_copy(
                    params_hbm.at[pl.ds(src_row + i * rc, rc)],
                    stage.at[pl.ds(i * rc, rc)],
                    gsem,
                )
            )
        scatters = []
        for i in range(nchunk):
            gathers[i].wait()
            scatters.append(
                pltpu.async_copy(
                    stage.at[pl.ds(i * rc, rc)],
                    out_hbm.at[pl.ds(dst_row + i * rc, rc)],
                    ssem,
                )
            )
        for s in scatters:
            s.wait()

    return run(probs, p, params_flat)


def kernel(probs, p, params, batch_size):
    K, B, D = params.shape
    info = plsc.get_sparse_core_info()
    NC = info.num_cores
    params_flat = params.reshape(K * B, D)
    return _mixture_select(probs, p, params_flat, K, B, D, NC)
